# baseline (speedup 1.0000x reference)
import functools

import jax
import jax.numpy as jnp
from jax import lax
from jax.experimental import pallas as pl
from jax.experimental.pallas import tpu as pltpu, tpu_sc as plsc

B = 4096
D = 64
DP = 2 * D
QV = 500000
L = 50
LP = 64
NC = 2
NS = 16
NW = NC * NS
RPW = B // NW
G = 2
NG = RPW // G
NV = D // 16
NBUF = 4


def _sc_pool(uid, hist_pad, hlen, tid, user_packed, item_packed):
    mesh = plsc.VectorSubcoreMesh(core_axis_name="c", subcore_axis_name="s")

    @functools.partial(
        pl.kernel,
        mesh=mesh,
        out_type=(
            jax.ShapeDtypeStruct((B, DP), jnp.float32),
            jax.ShapeDtypeStruct((B, D), jnp.float32),
            jax.ShapeDtypeStruct((B, DP), jnp.float32),
        ),
        scratch_types=(
            pltpu.VMEM((NG, G * LP), jnp.int32),
            pltpu.VMEM((NG, G * LP), jnp.float32),
            pltpu.VMEM((RPW,), jnp.int32),
            pltpu.VMEM((RPW,), jnp.int32),
            pltpu.VMEM((RPW + 16,), jnp.int32),
            tuple(pltpu.VMEM((G * LP, DP), jnp.float32)
                  for _ in range(NBUF)),
            pltpu.VMEM((RPW, D), jnp.float32),
            tuple(pltpu.SemaphoreType.DMA for _ in range(NBUF)),
            pltpu.SemaphoreType.DMA,
            pltpu.SemaphoreType.DMA,
        ),
    )
    def k(uid_h, hist_h, len_h, tid_h, ut_h, it_h, it4_h,
          ue_o, pool_o, ie_o,
          hidx, par, uidx, tidx, lenv, bufs, pooled_v,
          sems, sem_u, sem_t):
        wid = lax.axis_index("s") * NC + lax.axis_index("c")
        base = wid * RPW
        del it4_h
        pltpu.sync_copy(hist_h.at[pl.ds(wid * NG, NG)], hidx)
        pltpu.sync_copy(uid_h.at[pl.ds(base, RPW)], uidx)
        pltpu.sync_copy(tid_h.at[pl.ds(base, RPW)], tidx)
        pltpu.sync_copy(len_h.at[pl.ds(base, RPW)], lenv.at[pl.ds(0, RPW)])

        for q in range(RPW // 16):
            s = pl.ds(q * 16, 16)
            uidx[s] = lax.shift_right_logical(uidx[s], 1)
            tidx[s] = lax.shift_right_logical(tidx[s], 1)

        def split_body(gi, carry):
            for q in range(G * LP // 16):
                s = pl.ds(q * 16, 16)
                v = hidx[gi, s]
                par[gi, s] = (v & 1).astype(jnp.float32)
                hidx[gi, s] = lax.shift_right_logical(v, 1)
            return carry

        lax.fori_loop(0, NG, split_body, 0, unroll=1)

        def start_group(g, b):
            for q in range(G * LP // 16):
                s = pl.ds(q * 16, 16)
                pltpu.async_copy(it_h.at[hidx[g, s]], bufs[b].at[s], sems[b])

        def wait_group(g, b):
            for q in range(G * LP // 16):
                s = pl.ds(q * 16, 16)
                pltpu.make_async_copy(
                    it_h.at[hidx[g, s]], bufs[b].at[s], sems[b]).wait()

        for b in range(NBUF):
            start_group(b, b)

        def group(g, b):
            buf = bufs[b]
            sem = sems[b]
            wait_group(g, b)
            for r in range(G):
                il = g * G + r
                len_s = lenv[pl.ds(il, 16)][0]
                accs = [jnp.zeros((16,), jnp.float32) for _ in range(NV)]
                pv16 = None
                for j in range(L):
                    q, lane = divmod(j, 16)
                    if lane == 0:
                        pv16 = par[g, pl.ds(r * LP + q * 16, 16)]
                    p_s = pv16[lane]
                    m_s = jnp.minimum(jnp.maximum(len_s - j, 0),
                                      1).astype(jnp.float32)
                    s1 = m_s * p_s
                    s0 = m_s - s1
                    s0v = jnp.full((16,), s0, jnp.float32)
                    s1v = jnp.full((16,), s1, jnp.float32)
                    for c in range(NV):
                        a = buf[r * LP + j, pl.ds(c * 16, 16)]
                        bb = buf[r * LP + j, pl.ds(D + c * 16, 16)]
                        accs[c] = accs[c] + a * s0v + bb * s1v
                denom = jnp.full((16,), len_s, jnp.int32).astype(
                    jnp.float32) + 1e-8
                for c in range(NV):
                    pooled_v[il, pl.ds(c * 16, 16)] = accs[c] / denom
            @pl.when(g + NBUF < NG)
            def _():
                start_group(g + NBUF, b)

        def body(i, carry):
            for b in range(NBUF):
                group(i * NBUF + b, b)
            return carry

        lax.fori_loop(0, NG // NBUF, body, 0, unroll=1)

        pltpu.async_copy(ut_h.at[uidx], bufs[0], sem_u)
        pltpu.async_copy(it_h.at[tidx], bufs[1], sem_t)
        pltpu.make_async_copy(ut_h.at[uidx], bufs[0], sem_u).wait()
        pltpu.make_async_copy(it_h.at[tidx], bufs[1], sem_t).wait()
        pltpu.sync_copy(pooled_v, pool_o.at[pl.ds(base, RPW)])
        pltpu.sync_copy(bufs[0], ue_o.at[pl.ds(base, RPW)])
        pltpu.sync_copy(bufs[1], ie_o.at[pl.ds(base, RPW)])

    return k(uid, hist_pad, hlen, tid, user_packed, item_packed,
             item_packed.reshape(8 * QV, 16))


def _mlp_body(ue, pool, ie, uids, tids, w1u, w1p, b1, w2, b2, ur_o, ir_o):
    usel = (uids[...] & 1) == 1
    uemb = jnp.where(usel, ue[:, D:], ue[:, :D])
    h1 = jnp.dot(uemb, w1u[...], preferred_element_type=jnp.float32)
    h1 = h1 + jnp.dot(pool[...], w1p[...], preferred_element_type=jnp.float32)
    h1 = jnp.maximum(h1 + b1[...], 0.0)
    h2 = jnp.dot(h1, w2[...], preferred_element_type=jnp.float32)
    h2 = jnp.maximum(h2 + b2[...], 0.0)
    n = jnp.sqrt(jnp.sum(h2 * h2, axis=1, keepdims=True))
    ur_o[...] = h2 / jnp.maximum(n, 1e-12)
    tsel = (tids[...] & 1) == 1
    iev = jnp.where(tsel, ie[:, D:], ie[:, :D])
    ni = jnp.sqrt(jnp.sum(iev * iev, axis=1, keepdims=True))
    ir_o[...] = iev / jnp.maximum(ni, 1e-12)


def _mlp(ue, pool, ie, uids, tids, w1u, w1p, b1, w2, b2):
    T = 512
    grid = (B // T,)
    return pl.pallas_call(
        _mlp_body,
        grid=grid,
        in_specs=[
            pl.BlockSpec((T, DP), lambda i: (i, 0)),
            pl.BlockSpec((T, D), lambda i: (i, 0)),
            pl.BlockSpec((T, DP), lambda i: (i, 0)),
            pl.BlockSpec((T, 1), lambda i: (i, 0)),
            pl.BlockSpec((T, 1), lambda i: (i, 0)),
            pl.BlockSpec((D, 128), lambda i: (0, 0)),
            pl.BlockSpec((D, 128), lambda i: (0, 0)),
            pl.BlockSpec((1, 128), lambda i: (0, 0)),
            pl.BlockSpec((128, D), lambda i: (0, 0)),
            pl.BlockSpec((1, D), lambda i: (0, 0)),
        ],
        out_specs=[
            pl.BlockSpec((T, D), lambda i: (i, 0)),
            pl.BlockSpec((T, D), lambda i: (i, 0)),
        ],
        out_shape=[
            jax.ShapeDtypeStruct((B, D), jnp.float32),
            jax.ShapeDtypeStruct((B, D), jnp.float32),
        ],
    )(ue, pool, ie, uids, tids, w1u, w1p, b1, w2, b2)


def kernel(user_id, hist_items, hist_len, target_item, user_table, item_table,
           W1, b1, W2, b2):
    uid = user_id.astype(jnp.int32)
    tid = target_item.astype(jnp.int32)
    hist_pad = jnp.concatenate(
        [hist_items.astype(jnp.int32), jnp.zeros((B, LP - L), jnp.int32)],
        axis=1).reshape(B // G, G * LP)
    ut_p = user_table.reshape(QV, DP)
    it_p = item_table.reshape(QV, DP)
    ue, pool, ie = _sc_pool(uid, hist_pad, hist_len.astype(jnp.int32), tid,
                            ut_p, it_p)
    ur, ir = _mlp(ue, pool, ie, uid.reshape(B, 1), tid.reshape(B, 1),
                  W1[:D], W1[D:], b1.reshape(1, -1), W2, b2.reshape(1, -1))
    return ur, ir


# speedup vs baseline: 4.0140x; 4.0140x over previous
import functools

import jax
import jax.numpy as jnp
from jax import lax
from jax.experimental import pallas as pl
from jax.experimental.pallas import tpu as pltpu, tpu_sc as plsc

B = 4096
D = 64
L = 50
LP = 64
NC = 2
NS = 16
NW = NC * NS
RPW = B // NW
G = 2
NG = RPW // G
NV = D // 16
NBUF = 4
SENT = -1


def _sc_pool(uid, hist_pad, hlen, tid, user_table, item_table):
    mesh = plsc.VectorSubcoreMesh(core_axis_name="c", subcore_axis_name="s")

    @functools.partial(
        pl.kernel,
        mesh=mesh,
        compiler_params=pltpu.CompilerParams(use_tc_tiling_on_sc=False),
        out_type=(
            jax.ShapeDtypeStruct((B, D), jnp.float32),
            jax.ShapeDtypeStruct((B, D), jnp.float32),
            jax.ShapeDtypeStruct((B, D), jnp.float32),
        ),
        scratch_types=(
            pltpu.VMEM((NG, G * LP), jnp.int32),
            pltpu.VMEM((RPW,), jnp.int32),
            pltpu.VMEM((RPW,), jnp.int32),
            pltpu.VMEM((RPW + 16,), jnp.int32),
            tuple(pltpu.VMEM((G * LP, D), jnp.float32)
                  for _ in range(NBUF)),
            pltpu.VMEM((RPW, D), jnp.float32),
            pltpu.VMEM((RPW, D), jnp.float32),
            pltpu.VMEM((RPW, D), jnp.float32),
            tuple(pltpu.SemaphoreType.DMA for _ in range(NBUF)),
            pltpu.SemaphoreType.DMA,
            pltpu.SemaphoreType.DMA,
        ),
    )
    def k(uid_h, hist_h, len_h, tid_h, ut_h, it_h,
          ue_o, pool_o, ie_o,
          hidx, uidx, tidx, lenv, bufs, urows, irows, pooled_v,
          sems, sem_u, sem_t):
        wid = lax.axis_index("s") * NC + lax.axis_index("c")
        base = wid * RPW
        pltpu.sync_copy(hist_h.at[pl.ds(wid * NG, NG)], hidx)
        pltpu.sync_copy(uid_h.at[pl.ds(base, RPW)], uidx)
        pltpu.sync_copy(tid_h.at[pl.ds(base, RPW)], tidx)
        pltpu.sync_copy(len_h.at[pl.ds(base, RPW)], lenv.at[pl.ds(0, RPW)])
        pltpu.async_copy(ut_h.at[uidx], urows, sem_u)
        pltpu.async_copy(it_h.at[tidx], irows, sem_t)

        jvs = [jnp.arange(16, dtype=jnp.int32) + 16 * q
               for q in range(LP // 16)]

        def mask_body(gi, carry):
            for r in range(G):
                il = gi * G + r
                len_splat = jnp.full((16,), lenv[pl.ds(il, 16)][0], jnp.int32)
                for q in range(LP // 16):
                    s = pl.ds(r * LP + q * 16, 16)
                    sel = jnp.minimum(jnp.maximum(len_splat - jvs[q], 0), 1)
                    v = hidx[gi, s]
                    hidx[gi, s] = v * sel + SENT * (1 - sel)
            return carry

        lax.fori_loop(0, NG, mask_body, 0, unroll=1)

        def start_group(g, b):
            pltpu.async_copy(
                it_h.at[plsc.Indices(hidx.at[g], ignored_value=SENT)],
                bufs[b], sems[b])

        def wait_group(g, b):
            pltpu.make_async_copy(
                it_h.at[plsc.Indices(hidx.at[g], ignored_value=SENT)],
                bufs[b], sems[b]).wait()

        for b in range(NBUF):
            start_group(b, b)

        def group(g, b):
            buf = bufs[b]
            wait_group(g, b)
            for r in range(G):
                il = g * G + r
                len_splat = jnp.full((16,), lenv[pl.ds(il, 16)][0], jnp.int32)
                accs = [jnp.zeros((16,), jnp.float32) for _ in range(NV)]
                for j in range(L):
                    m = jnp.minimum(jnp.maximum(len_splat - j, 0),
                                    1).astype(jnp.float32)
                    for c in range(NV):
                        row = buf[r * LP + j, pl.ds(c * 16, 16)]
                        accs[c] = accs[c] + row * m
                denom = len_splat.astype(jnp.float32) + 1e-8
                for c in range(NV):
                    pooled_v[il, pl.ds(c * 16, 16)] = accs[c] / denom
            @pl.when(g + NBUF < NG)
            def _():
                start_group(g + NBUF, b)

        def body(i, carry):
            for b in range(NBUF):
                group(i * NBUF + b, b)
            return carry

        lax.fori_loop(0, NG // NBUF, body, 0, unroll=1)

        pltpu.make_async_copy(ut_h.at[uidx], urows, sem_u).wait()
        pltpu.make_async_copy(it_h.at[tidx], irows, sem_t).wait()
        pltpu.sync_copy(pooled_v, pool_o.at[pl.ds(base, RPW)])
        pltpu.sync_copy(urows, ue_o.at[pl.ds(base, RPW)])
        pltpu.sync_copy(irows, ie_o.at[pl.ds(base, RPW)])

    return k(uid, hist_pad, hlen, tid, user_table, item_table)


def _mlp_body(ue, pool, ie, w1u, w1p, b1, w2, b2, ur_o, ir_o):
    h1 = jnp.dot(ue[...], w1u[...], preferred_element_type=jnp.float32)
    h1 = h1 + jnp.dot(pool[...], w1p[...], preferred_element_type=jnp.float32)
    h1 = jnp.maximum(h1 + b1[...], 0.0)
    h2 = jnp.dot(h1, w2[...], preferred_element_type=jnp.float32)
    h2 = jnp.maximum(h2 + b2[...], 0.0)
    n = jnp.sqrt(jnp.sum(h2 * h2, axis=1, keepdims=True))
    ur_o[...] = h2 / jnp.maximum(n, 1e-12)
    iev = ie[...]
    ni = jnp.sqrt(jnp.sum(iev * iev, axis=1, keepdims=True))
    ir_o[...] = iev / jnp.maximum(ni, 1e-12)


def _mlp(ue, pool, ie, w1u, w1p, b1, w2, b2):
    T = 512
    grid = (B // T,)
    return pl.pallas_call(
        _mlp_body,
        grid=grid,
        in_specs=[
            pl.BlockSpec((T, D), lambda i: (i, 0)),
            pl.BlockSpec((T, D), lambda i: (i, 0)),
            pl.BlockSpec((T, D), lambda i: (i, 0)),
            pl.BlockSpec((D, 128), lambda i: (0, 0)),
            pl.BlockSpec((D, 128), lambda i: (0, 0)),
            pl.BlockSpec((1, 128), lambda i: (0, 0)),
            pl.BlockSpec((128, D), lambda i: (0, 0)),
            pl.BlockSpec((1, D), lambda i: (0, 0)),
        ],
        out_specs=[
            pl.BlockSpec((T, D), lambda i: (i, 0)),
            pl.BlockSpec((T, D), lambda i: (i, 0)),
        ],
        out_shape=[
            jax.ShapeDtypeStruct((B, D), jnp.float32),
            jax.ShapeDtypeStruct((B, D), jnp.float32),
        ],
    )(ue, pool, ie, w1u, w1p, b1, w2, b2)


def kernel(user_id, hist_items, hist_len, target_item, user_table, item_table,
           W1, b1, W2, b2):
    uid = user_id.astype(jnp.int32)
    hist_pad = jnp.concatenate(
        [hist_items.astype(jnp.int32), jnp.zeros((B, LP - L), jnp.int32)],
        axis=1).reshape(B // G, G * LP)
    ue, pool, ie = _sc_pool(uid, hist_pad, hist_len.astype(jnp.int32),
                            target_item.astype(jnp.int32),
                            user_table, item_table)
    ur, ir = _mlp(ue, pool, ie, W1[:D], W1[D:], b1.reshape(1, -1),
                  W2, b2.reshape(1, -1))
    return ur, ir


# speedup vs baseline: 4.2672x; 1.0631x over previous
import functools

import jax
import jax.numpy as jnp
from jax import lax
from jax.experimental import pallas as pl
from jax.experimental.pallas import tpu as pltpu, tpu_sc as plsc

B = 4096
D = 64
DP = 2 * D
L = 50
LP = 64
NC = 2
NS = 16
NW = NC * NS
RPW = B // NW
G = 2
NG = RPW // G
NV = D // 16
NBUF = 4
SENT = -1


def _sc_pool(uid, hist_pad, hlen, tid, user_table, item_table):
    mesh = plsc.VectorSubcoreMesh(core_axis_name="c", subcore_axis_name="s")

    @functools.partial(
        pl.kernel,
        mesh=mesh,
        compiler_params=pltpu.CompilerParams(use_tc_tiling_on_sc=False),
        out_type=(
            jax.ShapeDtypeStruct((B, DP), jnp.float32),
            jax.ShapeDtypeStruct((B, D), jnp.float32),
            jax.ShapeDtypeStruct((B, DP), jnp.float32),
        ),
        scratch_types=(
            pltpu.VMEM((NG, G * LP), jnp.int32),
            pltpu.VMEM((RPW,), jnp.int32),
            pltpu.VMEM((RPW,), jnp.int32),
            pltpu.VMEM((RPW + 16,), jnp.int32),
            tuple(pltpu.VMEM((G * LP, DP), jnp.float32)
                  for _ in range(NBUF)),
            pltpu.VMEM((RPW, DP), jnp.float32),
            pltpu.VMEM((RPW, DP), jnp.float32),
            pltpu.VMEM((RPW, D), jnp.float32),
            tuple(pltpu.SemaphoreType.DMA for _ in range(NBUF)),
            pltpu.SemaphoreType.DMA,
            pltpu.SemaphoreType.DMA,
        ),
    )
    def k(uid_h, hist_h, len_h, tid_h, ut_h, it_h,
          ue_o, pool_o, ie_o,
          hidx, uidx, tidx, lenv, bufs, urows, irows, pooled_v,
          sems, sem_u, sem_t):
        wid = lax.axis_index("s") * NC + lax.axis_index("c")
        base = wid * RPW
        pltpu.sync_copy(hist_h.at[pl.ds(wid * NG, NG)], hidx)
        pltpu.sync_copy(uid_h.at[pl.ds(base, RPW)], uidx)
        pltpu.sync_copy(tid_h.at[pl.ds(base, RPW)], tidx)
        pltpu.sync_copy(len_h.at[pl.ds(base, RPW)], lenv.at[pl.ds(0, RPW)])
        pltpu.async_copy(ut_h.at[plsc.Indices(uidx, ignored_value=SENT)],
                         urows, sem_u)
        pltpu.async_copy(it_h.at[plsc.Indices(tidx, ignored_value=SENT)],
                         irows, sem_t)

        jvs = [jnp.arange(16, dtype=jnp.int32) + 16 * q
               for q in range(LP // 16)]

        def mask_body(gi, carry):
            for r in range(G):
                il = gi * G + r
                len_splat = jnp.full((16,), lenv[pl.ds(il, 16)][0], jnp.int32)
                for q in range(LP // 16):
                    s = pl.ds(r * LP + q * 16, 16)
                    sel = jnp.minimum(jnp.maximum(len_splat - jvs[q], 0), 1)
                    v = hidx[gi, s]
                    hidx[gi, s] = v * sel + SENT * (1 - sel)
            return carry

        lax.fori_loop(0, NG, mask_body, 0, unroll=1)

        def start_group(g, b):
            pltpu.async_copy(
                it_h.at[plsc.Indices(hidx.at[g], ignored_value=SENT)],
                bufs[b], sems[b])

        def wait_group(g, b):
            pltpu.make_async_copy(
                it_h.at[plsc.Indices(hidx.at[g], ignored_value=SENT)],
                bufs[b], sems[b]).wait()

        for b in range(NBUF):
            start_group(b, b)

        def group(g, b):
            buf = bufs[b]
            wait_group(g, b)
            for r in range(G):
                il = g * G + r
                len_splat = jnp.full((16,), lenv[pl.ds(il, 16)][0], jnp.int32)
                accs = [jnp.zeros((16,), jnp.float32) for _ in range(NV)]
                for j in range(L):
                    m = jnp.minimum(jnp.maximum(len_splat - j, 0),
                                    1).astype(jnp.float32)
                    for c in range(NV):
                        row = buf[r * LP + j, pl.ds(c * 16, 16)]
                        accs[c] = accs[c] + row * m
                denom = len_splat.astype(jnp.float32) + 1e-8
                for c in range(NV):
                    pooled_v[il, pl.ds(c * 16, 16)] = accs[c] / denom
            @pl.when(g + NBUF < NG)
            def _():
                start_group(g + NBUF, b)

        def body(i, carry):
            for b in range(NBUF):
                group(i * NBUF + b, b)
            return carry

        lax.fori_loop(0, NG // NBUF, body, 0, unroll=1)

        pltpu.make_async_copy(
            ut_h.at[plsc.Indices(uidx, ignored_value=SENT)],
            urows, sem_u).wait()
        pltpu.make_async_copy(
            it_h.at[plsc.Indices(tidx, ignored_value=SENT)],
            irows, sem_t).wait()
        pltpu.sync_copy(pooled_v, pool_o.at[pl.ds(base, RPW)])
        pltpu.sync_copy(urows, ue_o.at[pl.ds(base, RPW)])
        pltpu.sync_copy(irows, ie_o.at[pl.ds(base, RPW)])

    return k(uid, hist_pad, hlen, tid, user_table, item_table)


def _mlp_body(ue, pool, ie, w1u, w1p, b1, w2, b2, ur_o, ir_o):
    h1 = jnp.dot(ue[:, :D], w1u[...], preferred_element_type=jnp.float32)
    h1 = h1 + jnp.dot(pool[...], w1p[...], preferred_element_type=jnp.float32)
    h1 = jnp.maximum(h1 + b1[...], 0.0)
    h2 = jnp.dot(h1, w2[...], preferred_element_type=jnp.float32)
    h2 = jnp.maximum(h2 + b2[...], 0.0)
    n = jnp.sqrt(jnp.sum(h2 * h2, axis=1, keepdims=True))
    ur_o[...] = h2 / jnp.maximum(n, 1e-12)
    iev = ie[:, :D]
    ni = jnp.sqrt(jnp.sum(iev * iev, axis=1, keepdims=True))
    ir_o[...] = iev / jnp.maximum(ni, 1e-12)


def _mlp(ue, pool, ie, w1u, w1p, b1, w2, b2):
    T = 512
    grid = (B // T,)
    return pl.pallas_call(
        _mlp_body,
        grid=grid,
        in_specs=[
            pl.BlockSpec((T, DP), lambda i: (i, 0)),
            pl.BlockSpec((T, D), lambda i: (i, 0)),
            pl.BlockSpec((T, DP), lambda i: (i, 0)),
            pl.BlockSpec((D, 128), lambda i: (0, 0)),
            pl.BlockSpec((D, 128), lambda i: (0, 0)),
            pl.BlockSpec((1, 128), lambda i: (0, 0)),
            pl.BlockSpec((128, D), lambda i: (0, 0)),
            pl.BlockSpec((1, D), lambda i: (0, 0)),
        ],
        out_specs=[
            pl.BlockSpec((T, D), lambda i: (i, 0)),
            pl.BlockSpec((T, D), lambda i: (i, 0)),
        ],
        out_shape=[
            jax.ShapeDtypeStruct((B, D), jnp.float32),
            jax.ShapeDtypeStruct((B, D), jnp.float32),
        ],
    )(ue, pool, ie, w1u, w1p, b1, w2, b2)


def kernel(user_id, hist_items, hist_len, target_item, user_table, item_table,
           W1, b1, W2, b2):
    uid = user_id.astype(jnp.int32)
    hist_pad = jnp.concatenate(
        [hist_items.astype(jnp.int32), jnp.zeros((B, LP - L), jnp.int32)],
        axis=1).reshape(B // G, G * LP)
    ut_pad = jnp.pad(user_table, ((0, 0), (0, DP - D)))
    it_pad = jnp.pad(item_table, ((0, 0), (0, DP - D)))
    ue, pool, ie = _sc_pool(uid, hist_pad, hist_len.astype(jnp.int32),
                            target_item.astype(jnp.int32),
                            ut_pad, it_pad)
    ur, ir = _mlp(ue, pool, ie, W1[:D], W1[D:], b1.reshape(1, -1),
                  W2, b2.reshape(1, -1))
    return ur, ir


# speedup vs baseline: 4.5568x; 1.0679x over previous
import functools

import jax
import jax.numpy as jnp
from jax import lax
from jax.experimental import pallas as pl
from jax.experimental.pallas import tpu as pltpu, tpu_sc as plsc

B = 4096
D = 64
DP = 2 * D
L = 50
LP = 64
NC = 2
NS = 16
NW = NC * NS
RPW = B // NW
G = 2
NG = RPW // G
NV = D // 16
NBUF = 4
SENT = -1
URING = 8


def _sc_pool(hist_pad, hlen, tid, item_table):
    mesh = plsc.VectorSubcoreMesh(core_axis_name="c", subcore_axis_name="s")

    @functools.partial(
        pl.kernel,
        mesh=mesh,
        compiler_params=pltpu.CompilerParams(use_tc_tiling_on_sc=False),
        out_type=(
            jax.ShapeDtypeStruct((B, D), jnp.float32),
            jax.ShapeDtypeStruct((B, DP), jnp.float32),
        ),
        scratch_types=(
            pltpu.VMEM((NG, G * LP), jnp.int32),
            pltpu.VMEM((RPW,), jnp.int32),
            pltpu.VMEM((RPW + 16,), jnp.int32),
            tuple(pltpu.VMEM((G * LP, DP), jnp.float32)
                  for _ in range(NBUF)),
            pltpu.VMEM((RPW, DP), jnp.float32),
            pltpu.VMEM((RPW, D), jnp.float32),
            tuple(pltpu.SemaphoreType.DMA for _ in range(NBUF)),
            pltpu.SemaphoreType.DMA,
        ),
    )
    def k(hist_h, len_h, tid_h, it_h,
          pool_o, ie_o,
          hidx, tidx, lenv, bufs, irows, pooled_v,
          sems, sem_t):
        wid = lax.axis_index("s") * NC + lax.axis_index("c")
        base = wid * RPW
        pltpu.sync_copy(hist_h.at[pl.ds(wid * NG, NG)], hidx)
        pltpu.sync_copy(tid_h.at[pl.ds(base, RPW)], tidx)
        pltpu.sync_copy(len_h.at[pl.ds(base, RPW)], lenv.at[pl.ds(0, RPW)])
        pltpu.async_copy(it_h.at[plsc.Indices(tidx, ignored_value=SENT)],
                         irows, sem_t)

        jvs = [jnp.arange(16, dtype=jnp.int32) + 16 * q
               for q in range(LP // 16)]

        def mask_body(gi, carry):
            for r in range(G):
                il = gi * G + r
                len_splat = jnp.full((16,), lenv[pl.ds(il, 16)][0], jnp.int32)
                for q in range(LP // 16):
                    s = pl.ds(r * LP + q * 16, 16)
                    sel = jnp.minimum(jnp.maximum(len_splat - jvs[q], 0), 1)
                    v = hidx[gi, s]
                    hidx[gi, s] = v * sel + SENT * (1 - sel)
            return carry

        lax.fori_loop(0, NG, mask_body, 0, unroll=1)

        def start_group(g, b):
            pltpu.async_copy(
                it_h.at[plsc.Indices(hidx.at[g], ignored_value=SENT)],
                bufs[b], sems[b])

        def wait_group(g, b):
            pltpu.make_async_copy(
                it_h.at[plsc.Indices(hidx.at[g], ignored_value=SENT)],
                bufs[b], sems[b]).wait()

        for b in range(NBUF):
            start_group(b, b)

        def group(g, b):
            buf = bufs[b]
            wait_group(g, b)
            for r in range(G):
                il = g * G + r
                len_splat = jnp.full((16,), lenv[pl.ds(il, 16)][0], jnp.int32)
                accs = [jnp.zeros((16,), jnp.float32) for _ in range(NV)]
                for j in range(L):
                    m = jnp.minimum(jnp.maximum(len_splat - j, 0),
                                    1).astype(jnp.float32)
                    for c in range(NV):
                        row = buf[r * LP + j, pl.ds(c * 16, 16)]
                        accs[c] = accs[c] + row * m
                denom = len_splat.astype(jnp.float32) + 1e-8
                for c in range(NV):
                    pooled_v[il, pl.ds(c * 16, 16)] = accs[c] / denom
            @pl.when(g + NBUF < NG)
            def _():
                start_group(g + NBUF, b)

        def body(i, carry):
            for b in range(NBUF):
                group(i * NBUF + b, b)
            return carry

        lax.fori_loop(0, NG // NBUF, body, 0, unroll=1)

        pltpu.make_async_copy(
            it_h.at[plsc.Indices(tidx, ignored_value=SENT)],
            irows, sem_t).wait()
        pltpu.sync_copy(pooled_v, pool_o.at[pl.ds(base, RPW)])
        pltpu.sync_copy(irows, ie_o.at[pl.ds(base, RPW)])

    return k(hist_pad, hlen, tid, item_table)


def _user_blk(utT, uid_s, r, blk, sem):
    idx = uid_s[r]
    base = pl.multiple_of((idx // 128) * 128, 128)
    return pltpu.make_async_copy(utT.at[:, pl.ds(base, 128)], blk, sem)


def _user_gather_body(uid_s, utT, outT, *scratch):
    blks = scratch[:URING]
    sems = scratch[URING:]
    lane = lax.broadcasted_iota(jnp.int32, (D, 128), 1)
    for b in range(URING):
        _user_blk(utT, uid_s, b, blks[b], sems[b]).start()

    def block_loop(k, carry):
        def inner(t, acc):
            for b in range(URING):
                j = t * URING + b
                r = k * 128 + j
                _user_blk(utT, uid_s, r, blks[b], sems[b]).wait()
                x = blks[b][...]
                col = uid_s[r] % 128
                col_v = jnp.sum(jnp.where(lane == col, x, 0.0),
                                axis=1, keepdims=True)
                acc = jnp.where(lane == j, col_v, acc)
                nxt = r + URING

                @pl.when(nxt < B)
                def _():
                    _user_blk(utT, uid_s, nxt, blks[b], sems[b]).start()
            return acc

        acc = lax.fori_loop(0, 128 // URING, inner,
                            jnp.zeros((D, 128), jnp.float32))
        outT[:, pl.ds(pl.multiple_of(k * 128, 128), 128)] = acc
        return carry

    lax.fori_loop(0, B // 128, block_loop, 0)


def _user_gather(uid, user_table_t):
    return pl.pallas_call(
        _user_gather_body,
        in_specs=[
            pl.BlockSpec(memory_space=pltpu.SMEM),
            pl.BlockSpec(memory_space=pl.ANY),
        ],
        out_specs=pl.BlockSpec(memory_space=pltpu.VMEM),
        out_shape=jax.ShapeDtypeStruct((D, B), jnp.float32),
        scratch_shapes=[pltpu.VMEM((D, 128), jnp.float32)] * URING
        + [pltpu.SemaphoreType.DMA] * URING,
    )(uid, user_table_t)


def _mlp_body(ue, pool, ie, w1u, w1p, b1, w2, b2, ur_o, ir_o):
    h1 = jnp.dot(ue[...], w1u[...], preferred_element_type=jnp.float32)
    h1 = h1 + jnp.dot(pool[...], w1p[...], preferred_element_type=jnp.float32)
    h1 = jnp.maximum(h1 + b1[...], 0.0)
    h2 = jnp.dot(h1, w2[...], preferred_element_type=jnp.float32)
    h2 = jnp.maximum(h2 + b2[...], 0.0)
    n = jnp.sqrt(jnp.sum(h2 * h2, axis=1, keepdims=True))
    ur_o[...] = h2 / jnp.maximum(n, 1e-12)
    iev = ie[:, :D]
    ni = jnp.sqrt(jnp.sum(iev * iev, axis=1, keepdims=True))
    ir_o[...] = iev / jnp.maximum(ni, 1e-12)


def _mlp(ue, pool, ie, w1u, w1p, b1, w2, b2):
    T = 512
    grid = (B // T,)
    return pl.pallas_call(
        _mlp_body,
        grid=grid,
        in_specs=[
            pl.BlockSpec((T, D), lambda i: (i, 0)),
            pl.BlockSpec((T, D), lambda i: (i, 0)),
            pl.BlockSpec((T, DP), lambda i: (i, 0)),
            pl.BlockSpec((D, 128), lambda i: (0, 0)),
            pl.BlockSpec((D, 128), lambda i: (0, 0)),
            pl.BlockSpec((1, 128), lambda i: (0, 0)),
            pl.BlockSpec((128, D), lambda i: (0, 0)),
            pl.BlockSpec((1, D), lambda i: (0, 0)),
        ],
        out_specs=[
            pl.BlockSpec((T, D), lambda i: (i, 0)),
            pl.BlockSpec((T, D), lambda i: (i, 0)),
        ],
        out_shape=[
            jax.ShapeDtypeStruct((B, D), jnp.float32),
            jax.ShapeDtypeStruct((B, D), jnp.float32),
        ],
    )(ue, pool, ie, w1u, w1p, b1, w2, b2)


def kernel(user_id, hist_items, hist_len, target_item, user_table, item_table,
           W1, b1, W2, b2):
    uid = user_id.astype(jnp.int32)
    hist_pad = jnp.concatenate(
        [hist_items.astype(jnp.int32), jnp.zeros((B, LP - L), jnp.int32)],
        axis=1).reshape(B // G, G * LP)
    it_pad = jnp.pad(item_table, ((0, 0), (0, DP - D)))
    pool, ie = _sc_pool(hist_pad, hist_len.astype(jnp.int32),
                        target_item.astype(jnp.int32), it_pad)
    ue = _user_gather(uid, user_table.T).T
    ur, ir = _mlp(ue, pool, ie, W1[:D], W1[D:], b1.reshape(1, -1),
                  W2, b2.reshape(1, -1))
    return ur, ir


# speedup vs baseline: 5.0786x; 1.1145x over previous
import functools

import jax
import jax.numpy as jnp
from jax import lax
from jax.experimental import pallas as pl
from jax.experimental.pallas import tpu as pltpu, tpu_sc as plsc

B = 4096
D = 64
DP = 2 * D
L = 50
LP = 64
NC = 2
NS = 16
NW = NC * NS
RPW = B // NW
G = 2
NG = RPW // G
NV = D // 16
NBUF = 4
SENT = -1
URING = 16


def _sc_pool(hist_pad, hlen, tid, item_table):
    mesh = plsc.VectorSubcoreMesh(core_axis_name="c", subcore_axis_name="s")

    @functools.partial(
        pl.kernel,
        mesh=mesh,
        compiler_params=pltpu.CompilerParams(use_tc_tiling_on_sc=False),
        out_type=(
            jax.ShapeDtypeStruct((B, D), jnp.float32),
            jax.ShapeDtypeStruct((B, DP), jnp.float32),
        ),
        scratch_types=(
            pltpu.VMEM((NG, G * LP), jnp.int32),
            pltpu.VMEM((RPW,), jnp.int32),
            pltpu.VMEM((RPW + 16,), jnp.int32),
            tuple(pltpu.VMEM((G * LP, DP), jnp.float32)
                  for _ in range(NBUF)),
            pltpu.VMEM((RPW, DP), jnp.float32),
            pltpu.VMEM((RPW, D), jnp.float32),
            tuple(pltpu.SemaphoreType.DMA for _ in range(NBUF)),
            pltpu.SemaphoreType.DMA,
        ),
    )
    def k(hist_h, len_h, tid_h, it_h,
          pool_o, ie_o,
          hidx, tidx, lenv, bufs, irows, pooled_v,
          sems, sem_t):
        wid = lax.axis_index("s") * NC + lax.axis_index("c")
        base = wid * RPW
        pltpu.sync_copy(hist_h.at[pl.ds(wid * NG, NG)], hidx)
        pltpu.sync_copy(tid_h.at[pl.ds(base, RPW)], tidx)
        pltpu.sync_copy(len_h.at[pl.ds(base, RPW)], lenv.at[pl.ds(0, RPW)])
        pltpu.async_copy(it_h.at[plsc.Indices(tidx, ignored_value=SENT)],
                         irows, sem_t)

        jvs = [jnp.arange(16, dtype=jnp.int32) + 16 * q
               for q in range(LP // 16)]

        def mask_body(gi, carry):
            for r in range(G):
                il = gi * G + r
                len_splat = jnp.full((16,), lenv[pl.ds(il, 16)][0], jnp.int32)
                for q in range(LP // 16):
                    s = pl.ds(r * LP + q * 16, 16)
                    sel = jnp.minimum(jnp.maximum(len_splat - jvs[q], 0), 1)
                    v = hidx[gi, s]
                    hidx[gi, s] = v * sel + SENT * (1 - sel)
            return carry

        lax.fori_loop(0, NG, mask_body, 0, unroll=1)

        def start_group(g, b):
            pltpu.async_copy(
                it_h.at[plsc.Indices(hidx.at[g], ignored_value=SENT)],
                bufs[b], sems[b])

        def wait_group(g, b):
            pltpu.make_async_copy(
                it_h.at[plsc.Indices(hidx.at[g], ignored_value=SENT)],
                bufs[b], sems[b]).wait()

        for b in range(NBUF):
            start_group(b, b)

        def group(g, b):
            buf = bufs[b]
            wait_group(g, b)
            for r in range(G):
                il = g * G + r
                len_splat = jnp.full((16,), lenv[pl.ds(il, 16)][0], jnp.int32)
                accs = [jnp.zeros((16,), jnp.float32) for _ in range(NV)]
                for j in range(L):
                    m = jnp.minimum(jnp.maximum(len_splat - j, 0),
                                    1).astype(jnp.float32)
                    for c in range(NV):
                        row = buf[r * LP + j, pl.ds(c * 16, 16)]
                        accs[c] = accs[c] + row * m
                denom = len_splat.astype(jnp.float32) + 1e-8
                for c in range(NV):
                    pooled_v[il, pl.ds(c * 16, 16)] = accs[c] / denom
            @pl.when(g + NBUF < NG)
            def _():
                start_group(g + NBUF, b)

        def body(i, carry):
            for b in range(NBUF):
                group(i * NBUF + b, b)
            return carry

        lax.fori_loop(0, NG // NBUF, body, 0, unroll=1)

        pltpu.make_async_copy(
            it_h.at[plsc.Indices(tidx, ignored_value=SENT)],
            irows, sem_t).wait()
        pltpu.sync_copy(pooled_v, pool_o.at[pl.ds(base, RPW)])
        pltpu.sync_copy(irows, ie_o.at[pl.ds(base, RPW)])

    return k(hist_pad, hlen, tid, item_table)


def _user_blk(utT, uid_s, r, blk, sem):
    idx = uid_s[r]
    base = pl.multiple_of((idx // 128) * 128, 128)
    return pltpu.make_async_copy(utT.at[:, pl.ds(base, 128)], blk, sem)


def _user_gather_body(uid_s, utT, outT, *scratch):
    blks = scratch[:URING]
    sems = scratch[URING:]
    lane = lax.broadcasted_iota(jnp.int32, (D, 128), 1)
    for b in range(URING):
        _user_blk(utT, uid_s, b, blks[b], sems[b]).start()

    def block_loop(k, carry):
        def inner(t, acc):
            for b in range(URING):
                j = t * URING + b
                r = k * 128 + j
                _user_blk(utT, uid_s, r, blks[b], sems[b]).wait()
                x = blks[b][...]
                col = uid_s[r] % 128
                col_v = jnp.sum(jnp.where(lane == col, x, 0.0),
                                axis=1, keepdims=True)
                acc = jnp.where(lane == j, col_v, acc)
                nxt = r + URING

                @pl.when(nxt < B)
                def _():
                    _user_blk(utT, uid_s, nxt, blks[b], sems[b]).start()
            return acc

        acc = lax.fori_loop(0, 128 // URING, inner,
                            jnp.zeros((D, 128), jnp.float32))
        outT[:, pl.ds(pl.multiple_of(k * 128, 128), 128)] = acc
        return carry

    lax.fori_loop(0, B // 128, block_loop, 0)


def _user_gather(uid, user_table_t):
    return pl.pallas_call(
        _user_gather_body,
        in_specs=[
            pl.BlockSpec(memory_space=pltpu.SMEM),
            pl.BlockSpec(memory_space=pl.ANY),
        ],
        out_specs=pl.BlockSpec(memory_space=pltpu.VMEM),
        out_shape=jax.ShapeDtypeStruct((D, B), jnp.float32),
        scratch_shapes=[pltpu.VMEM((D, 128), jnp.float32)] * URING
        + [pltpu.SemaphoreType.DMA] * URING,
    )(uid, user_table_t)


def _mlp_body(ue, pool, ie, w1u, w1p, b1, w2, b2, ur_o, ir_o):
    h1 = jnp.dot(ue[...], w1u[...], preferred_element_type=jnp.float32)
    h1 = h1 + jnp.dot(pool[...], w1p[...], preferred_element_type=jnp.float32)
    h1 = jnp.maximum(h1 + b1[...], 0.0)
    h2 = jnp.dot(h1, w2[...], preferred_element_type=jnp.float32)
    h2 = jnp.maximum(h2 + b2[...], 0.0)
    n = jnp.sqrt(jnp.sum(h2 * h2, axis=1, keepdims=True))
    ur_o[...] = h2 / jnp.maximum(n, 1e-12)
    iev = ie[:, :D]
    ni = jnp.sqrt(jnp.sum(iev * iev, axis=1, keepdims=True))
    ir_o[...] = iev / jnp.maximum(ni, 1e-12)


def _mlp(ue, pool, ie, w1u, w1p, b1, w2, b2):
    T = 512
    grid = (B // T,)
    return pl.pallas_call(
        _mlp_body,
        grid=grid,
        in_specs=[
            pl.BlockSpec((T, D), lambda i: (i, 0)),
            pl.BlockSpec((T, D), lambda i: (i, 0)),
            pl.BlockSpec((T, DP), lambda i: (i, 0)),
            pl.BlockSpec((D, 128), lambda i: (0, 0)),
            pl.BlockSpec((D, 128), lambda i: (0, 0)),
            pl.BlockSpec((1, 128), lambda i: (0, 0)),
            pl.BlockSpec((128, D), lambda i: (0, 0)),
            pl.BlockSpec((1, D), lambda i: (0, 0)),
        ],
        out_specs=[
            pl.BlockSpec((T, D), lambda i: (i, 0)),
            pl.BlockSpec((T, D), lambda i: (i, 0)),
        ],
        out_shape=[
            jax.ShapeDtypeStruct((B, D), jnp.float32),
            jax.ShapeDtypeStruct((B, D), jnp.float32),
        ],
    )(ue, pool, ie, w1u, w1p, b1, w2, b2)


def kernel(user_id, hist_items, hist_len, target_item, user_table, item_table,
           W1, b1, W2, b2):
    uid = user_id.astype(jnp.int32)
    hist_pad = jnp.concatenate(
        [hist_items.astype(jnp.int32), jnp.zeros((B, LP - L), jnp.int32)],
        axis=1).reshape(B // G, G * LP)
    it_pad = jnp.concatenate(
        [item_table, jnp.zeros((item_table.shape[0], DP - D), jnp.float32)],
        axis=1)
    pool, ie = _sc_pool(hist_pad, hist_len.astype(jnp.int32),
                        target_item.astype(jnp.int32), it_pad)
    ue = _user_gather(uid, user_table.T).T
    ur, ir = _mlp(ue, pool, ie, W1[:D], W1[D:], b1.reshape(1, -1),
                  W2, b2.reshape(1, -1))
    return ur, ir


# speedup vs baseline: 5.9071x; 1.1631x over previous
import functools

import jax
import jax.numpy as jnp
from jax import lax
from jax.experimental import pallas as pl
from jax.experimental.pallas import tpu as pltpu, tpu_sc as plsc

B = 4096
D = 64
DP = 2 * D
L = 50
LP = 64
NC = 2
NS = 16
NW = NC * NS
RPW = B // NW
G = 2
NG = RPW // G
NV = D // 16
NBUF = 4
SENT = -1
URING = 16


def _sc_pool(hist_pad, hlen, tid, item_table):
    mesh = plsc.VectorSubcoreMesh(core_axis_name="c", subcore_axis_name="s")

    @functools.partial(
        pl.kernel,
        mesh=mesh,
        compiler_params=pltpu.CompilerParams(use_tc_tiling_on_sc=False),
        out_type=(
            jax.ShapeDtypeStruct((B, D), jnp.float32),
            jax.ShapeDtypeStruct((B, DP), jnp.float32),
        ),
        scratch_types=(
            pltpu.VMEM((NG, G * LP), jnp.int32),
            pltpu.VMEM((RPW,), jnp.int32),
            pltpu.VMEM((RPW + 16,), jnp.int32),
            tuple(pltpu.VMEM((G * LP, DP), jnp.float32)
                  for _ in range(NBUF)),
            pltpu.VMEM((RPW, DP), jnp.float32),
            pltpu.VMEM((RPW, D), jnp.float32),
            tuple(pltpu.SemaphoreType.DMA for _ in range(NBUF)),
            pltpu.SemaphoreType.DMA,
        ),
    )
    def k(hist_h, len_h, tid_h, it_h,
          pool_o, ie_o,
          hidx, tidx, lenv, bufs, irows, pooled_v,
          sems, sem_t):
        wid = lax.axis_index("s") * NC + lax.axis_index("c")
        base = wid * RPW
        pltpu.sync_copy(hist_h.at[pl.ds(wid * NG, NG)], hidx)
        pltpu.sync_copy(tid_h.at[pl.ds(base, RPW)], tidx)
        pltpu.sync_copy(len_h.at[pl.ds(base, RPW)], lenv.at[pl.ds(0, RPW)])
        pltpu.async_copy(it_h.at[plsc.Indices(tidx, ignored_value=SENT)],
                         irows, sem_t)

        jvs = [jnp.arange(16, dtype=jnp.int32) + 16 * q
               for q in range(LP // 16)]

        def mask_body(gi, carry):
            for r in range(G):
                il = gi * G + r
                len_splat = jnp.full((16,), lenv[pl.ds(il, 16)][0], jnp.int32)
                for q in range(LP // 16):
                    s = pl.ds(r * LP + q * 16, 16)
                    sel = jnp.minimum(jnp.maximum(len_splat - jvs[q], 0), 1)
                    v = hidx[gi, s]
                    hidx[gi, s] = v * sel + SENT * (1 - sel)
            return carry

        lax.fori_loop(0, NG, mask_body, 0, unroll=1)

        def start_group(g, b):
            pltpu.async_copy(
                it_h.at[plsc.Indices(hidx.at[g], ignored_value=SENT)],
                bufs[b], sems[b])

        def wait_group(g, b):
            pltpu.make_async_copy(
                it_h.at[plsc.Indices(hidx.at[g], ignored_value=SENT)],
                bufs[b], sems[b]).wait()

        for b in range(NBUF):
            start_group(b, b)

        def group(g, b):
            buf = bufs[b]
            wait_group(g, b)
            for r in range(G):
                il = g * G + r
                len_splat = jnp.full((16,), lenv[pl.ds(il, 16)][0], jnp.int32)
                accs = [jnp.zeros((16,), jnp.float32) for _ in range(NV)]
                for j in range(L):
                    m = jnp.minimum(jnp.maximum(len_splat - j, 0),
                                    1).astype(jnp.float32)
                    for c in range(NV):
                        row = buf[r * LP + j, pl.ds(c * 16, 16)]
                        accs[c] = accs[c] + row * m
                denom = len_splat.astype(jnp.float32) + 1e-8
                for c in range(NV):
                    pooled_v[il, pl.ds(c * 16, 16)] = accs[c] / denom
            @pl.when(g + NBUF < NG)
            def _():
                start_group(g + NBUF, b)

        def body(i, carry):
            for b in range(NBUF):
                group(i * NBUF + b, b)
            return carry

        lax.fori_loop(0, NG // NBUF, body, 0, unroll=1)

        pltpu.make_async_copy(
            it_h.at[plsc.Indices(tidx, ignored_value=SENT)],
            irows, sem_t).wait()
        pltpu.sync_copy(pooled_v, pool_o.at[pl.ds(base, RPW)])
        pltpu.sync_copy(irows, ie_o.at[pl.ds(base, RPW)])

    return k(hist_pad, hlen, tid, item_table)


def _user_blk(utT, uid_s, r, blk, sem):
    idx = uid_s[r]
    base = pl.multiple_of((idx // 128) * 128, 128)
    return pltpu.make_async_copy(utT.at[:, pl.ds(base, 128)], blk, sem)


def _user_gather_body(uid_s, utT, outT, *scratch):
    blks = scratch[:URING]
    sems = scratch[URING:]
    lane = lax.broadcasted_iota(jnp.int32, (D, 128), 1)
    for b in range(URING):
        _user_blk(utT, uid_s, b, blks[b], sems[b]).start()

    def block_loop(k, carry):
        def inner(t, acc):
            for b in range(URING):
                j = t * URING + b
                r = k * 128 + j
                _user_blk(utT, uid_s, r, blks[b], sems[b]).wait()
                x = blks[b][...]
                col = uid_s[r] % 128
                col_v = jnp.sum(jnp.where(lane == col, x, 0.0),
                                axis=1, keepdims=True)
                acc = jnp.where(lane == j, col_v, acc)
                nxt = r + URING

                @pl.when(nxt < B)
                def _():
                    _user_blk(utT, uid_s, nxt, blks[b], sems[b]).start()
            return acc

        acc = lax.fori_loop(0, 128 // URING, inner,
                            jnp.zeros((D, 128), jnp.float32))
        outT[:, pl.ds(pl.multiple_of(k * 128, 128), 128)] = acc
        return carry

    lax.fori_loop(0, B // 128, block_loop, 0)


def _user_gather(uid, user_table_t):
    return pl.pallas_call(
        _user_gather_body,
        in_specs=[
            pl.BlockSpec(memory_space=pltpu.SMEM),
            pl.BlockSpec(memory_space=pl.ANY),
        ],
        out_specs=pl.BlockSpec(memory_space=pltpu.VMEM),
        out_shape=jax.ShapeDtypeStruct((D, B), jnp.float32),
        scratch_shapes=[pltpu.VMEM((D, 128), jnp.float32)] * URING
        + [pltpu.SemaphoreType.DMA] * URING,
    )(uid, user_table_t)


def _mlp_body(ue, pool, ie, w1u, w1p, b1, w2, b2, ur_o, ir_o):
    h1 = jnp.dot(ue[...], w1u[...], preferred_element_type=jnp.float32)
    h1 = h1 + jnp.dot(pool[...], w1p[...], preferred_element_type=jnp.float32)
    h1 = jnp.maximum(h1 + b1[...], 0.0)
    h2 = jnp.dot(h1, w2[...], preferred_element_type=jnp.float32)
    h2 = jnp.maximum(h2 + b2[...], 0.0)
    n = jnp.sqrt(jnp.sum(h2 * h2, axis=1, keepdims=True))
    ur_o[...] = h2 / jnp.maximum(n, 1e-12)
    iev = ie[:, :D]
    ni = jnp.sqrt(jnp.sum(iev * iev, axis=1, keepdims=True))
    ir_o[...] = iev / jnp.maximum(ni, 1e-12)


def _mlp(ue, pool, ie, w1u, w1p, b1, w2, b2):
    T = 512
    grid = (B // T,)
    return pl.pallas_call(
        _mlp_body,
        grid=grid,
        in_specs=[
            pl.BlockSpec((T, D), lambda i: (i, 0)),
            pl.BlockSpec((T, D), lambda i: (i, 0)),
            pl.BlockSpec((T, DP), lambda i: (i, 0)),
            pl.BlockSpec((D, 128), lambda i: (0, 0)),
            pl.BlockSpec((D, 128), lambda i: (0, 0)),
            pl.BlockSpec((1, 128), lambda i: (0, 0)),
            pl.BlockSpec((128, D), lambda i: (0, 0)),
            pl.BlockSpec((1, D), lambda i: (0, 0)),
        ],
        out_specs=[
            pl.BlockSpec((T, D), lambda i: (i, 0)),
            pl.BlockSpec((T, D), lambda i: (i, 0)),
        ],
        out_shape=[
            jax.ShapeDtypeStruct((B, D), jnp.float32),
            jax.ShapeDtypeStruct((B, D), jnp.float32),
        ],
    )(ue, pool, ie, w1u, w1p, b1, w2, b2)


def kernel(user_id, hist_items, hist_len, target_item, user_table, item_table,
           W1, b1, W2, b2):
    uid = user_id.astype(jnp.int32)
    hist_pad = jnp.concatenate(
        [hist_items.astype(jnp.int32), jnp.zeros((B, LP - L), jnp.int32)],
        axis=1).reshape(B // G, G * LP)
    ueT = _user_gather(uid, user_table.T)
    zpad = jnp.zeros((item_table.shape[0], DP - D), jnp.float32) + ueT[0, 0] * 0.0
    it_pad = jnp.concatenate([item_table, zpad], axis=1)
    pool, ie = _sc_pool(hist_pad, hist_len.astype(jnp.int32),
                        target_item.astype(jnp.int32), it_pad)
    ue = ueT.T
    ur, ir = _mlp(ue, pool, ie, W1[:D], W1[D:], b1.reshape(1, -1),
                  W2, b2.reshape(1, -1))
    return ur, ir
